# baseline (speedup 1.0000x reference)
import functools

import jax
import jax.numpy as jnp
from jax import lax
from jax.experimental import pallas as pl
from jax.experimental.pallas import tpu as pltpu
from jax.experimental.pallas import tpu_sc as plsc

NC = 2
NS = 16
LANES = 16
C = 80
SK = 8
UNR = 2 * SK


def _edge_linear(edge_attr, W, b, EP):
    E, ED = edge_attr.shape
    D = W.shape[1]
    B = 2000

    def body(a_ref, w_ref, b_ref, o_ref):
        o_ref[...] = (
            jnp.dot(a_ref[...], w_ref[...], preferred_element_type=jnp.float32)
            + b_ref[...]
        )

    return pl.pallas_call(
        body,
        grid=(E // B,),
        in_specs=[
            pl.BlockSpec((B, ED), lambda i: (i, 0)),
            pl.BlockSpec((ED, D), lambda i: (0, 0)),
            pl.BlockSpec((1, D), lambda i: (0, 0)),
        ],
        out_specs=pl.BlockSpec((B, D), lambda i: (i, 0)),
        out_shape=jax.ShapeDtypeStruct((EP, D), jnp.float32),
    )(edge_attr, W, b.reshape(1, D))


def _sc_message_pass(h, src2, dst2, t):
    N, D = h.shape
    EP = t.shape[0]
    NW = NC * NS
    EW = EP // NW
    NCH = EW // C
    NSC = NCH // SK
    NP = -(-(N + 1) // (NS * 8)) * (NS * 8)
    RPS = NP // NS
    JD = D // LANES
    assert NCH % UNR == 0 and RPS % 8 == 0
    NO = NCH // UNR

    mesh = plsc.VectorSubcoreMesh(core_axis_name="c", subcore_axis_name="s")

    @functools.partial(
        pl.kernel,
        out_type=jax.ShapeDtypeStruct((NC, NP, D), jnp.float32),
        mesh=mesh,
        scratch_types=(
            [pltpu.VMEM_SHARED((NP, D), jnp.float32)]
            + [pltpu.VMEM((SK, C), jnp.int32) for _ in range(2)]
            + [pltpu.VMEM((SK, C), jnp.int32) for _ in range(2)]
            + [pltpu.VMEM((C, D), jnp.float32) for _ in range(2)]
            + [pltpu.VMEM((C, D), jnp.float32) for _ in range(2)]
            + [pltpu.SemaphoreType.DMA for _ in range(6)]
        ),
    )
    def k(h_hbm, src_hbm, dst_hbm, t_hbm, out_hbm, acc_sh, *rest):
        srcv = rest[0:2]
        dstv = rest[2:4]
        tv = rest[4:6]
        rowsv = rest[6:8]
        sem_g, sem_s = rest[8], rest[9]
        sem_t = rest[10:12]
        sem_i = rest[12:14]

        cid = lax.axis_index("c")
        sid = lax.axis_index("s")
        wid = cid * NS + sid
        ebase = wid * EW
        cbase = wid * NCH
        row0 = sid * RPS

        def issue_idx(si, sl):
            row = cbase + si * SK
            pltpu.async_copy(src_hbm.at[pl.ds(row, SK)], srcv[sl], sem_i[sl])
            pltpu.async_copy(dst_hbm.at[pl.ds(row, SK)], dstv[sl], sem_i[sl])

        def wait_idx(sl):
            pltpu.make_async_copy(src_hbm.at[pl.ds(0, SK)], srcv[sl], sem_i[sl]).wait()
            pltpu.make_async_copy(dst_hbm.at[pl.ds(0, SK)], dstv[sl], sem_i[sl]).wait()

        def issue_t(c, sl):
            pltpu.async_copy(t_hbm.at[pl.ds(ebase + c * C, C)], tv[sl], sem_t[sl])

        def wait_t(sl):
            pltpu.make_async_copy(t_hbm.at[pl.ds(0, C)], tv[sl], sem_t[sl]).wait()

        def issue_gather(isl, krow, rsl):
            pltpu.async_copy(h_hbm.at[srcv[isl].at[krow]], rowsv[rsl], sem_g)

        def wait_gather(isl, krow, rsl):
            pltpu.make_async_copy(h_hbm.at[srcv[isl].at[krow]], rowsv[rsl], sem_g).wait()

        def issue_scatter(isl, krow, rsl):
            pltpu.async_copy(rowsv[rsl], acc_sh.at[dstv[isl].at[krow]], sem_s,
                             add=True)

        def drain_scatter(isl, krow, rsl):
            pltpu.make_async_copy(rowsv[rsl], acc_sh.at[dstv[isl].at[krow]],
                                  sem_s).wait()

        def compute(rsl):
            @pl.loop(0, C * JD, unroll=8)
            def _(g):
                i = g // JD
                jo = (g % JD) * LANES
                rowsv[rsl][i, pl.ds(jo, LANES)] = jnp.maximum(
                    rowsv[rsl][i, pl.ds(jo, LANES)]
                    + tv[rsl][i, pl.ds(jo, LANES)],
                    0.0,
                )

        zero = jnp.zeros((LANES,), jnp.float32)

        @pl.loop(0, C)
        def _(i):
            for j in range(JD):
                rowsv[0][i, pl.ds(j * LANES, LANES)] = zero

        @pl.loop(0, RPS // C)
        def _(r):
            pltpu.sync_copy(rowsv[0], acc_sh.at[pl.ds(row0 + r * C, C)])

        rem = RPS % C
        if rem:
            pltpu.sync_copy(
                rowsv[0].at[pl.ds(0, rem)],
                acc_sh.at[pl.ds(row0 + (RPS // C) * C, rem)],
            )
        plsc.subcore_barrier()

        issue_idx(0, 0)
        wait_idx(0)
        issue_t(0, 0)
        issue_t(1, 1)
        issue_gather(0, 0, 0)

        @pl.loop(0, NO)
        def _(o):
            for b in range(UNR):
                c = o * UNR + b
                rb = b % 2
                isl = (b // SK) % 2
                krow = b % SK
                isl1 = ((b + 1) // SK) % 2
                krow1 = (b + 1) % SK

                wait_t(rb)
                wait_gather(isl, krow, rb)
                compute(rb)

                if b > 0:
                    drain_scatter((b - 1) // SK % 2, (b - 1) % SK, 1 - rb)
                else:
                    @pl.when(o > 0)
                    def _():
                        drain_scatter((UNR - 1) // SK % 2, SK - 1, 1 - rb)

                if b < UNR - 2:
                    issue_t(c + 2, rb)
                else:
                    @pl.when(o < NO - 1)
                    def _():
                        issue_t(c + 2, rb)

                if krow == 1:
                    if b // SK == 0:
                        issue_idx(o * 2 + 1, 1)
                    else:
                        @pl.when(o < NO - 1)
                        def _():
                            issue_idx(o * 2 + 2, 0)

                if b == UNR - 1:
                    @pl.when(o < NO - 1)
                    def _():
                        wait_idx(0)
                        issue_gather(0, 0, 1 - rb)
                else:
                    if krow1 == 0:
                        wait_idx(isl1)
                    issue_gather(isl1, krow1, 1 - rb)

                issue_scatter(isl, krow, rb)

        drain_scatter((UNR - 1) // SK % 2, SK - 1, (NCH - 1) % 2)

        plsc.subcore_barrier()
        pltpu.sync_copy(
            acc_sh.at[pl.ds(row0, RPS)], out_hbm.at[cid, pl.ds(row0, RPS)]
        )

    return k(h, src2, dst2, t)


def _node_update(h, acc, W_nn, b_nn):
    N, D = h.shape
    B = 1000

    def body(h_ref, a_ref, w_ref, b_ref, o_ref):
        s = h_ref[...] + a_ref[0] + a_ref[1]
        z = jnp.dot(s, w_ref[...], preferred_element_type=jnp.float32) + b_ref[...]
        o_ref[...] = jnp.where(z >= 0, z, 0.01 * z)

    return pl.pallas_call(
        body,
        grid=(N // B,),
        in_specs=[
            pl.BlockSpec((B, D), lambda i: (i, 0)),
            pl.BlockSpec((NC, B, D), lambda i: (0, i, 0)),
            pl.BlockSpec((D, D), lambda i: (0, 0)),
            pl.BlockSpec((1, D), lambda i: (0, 0)),
        ],
        out_specs=pl.BlockSpec((B, D), lambda i: (i, 0)),
        out_shape=jax.ShapeDtypeStruct((N, D), jnp.float32),
    )(h, acc, W_nn, b_nn.reshape(1, D))


def _final(h, acc, W_nn, b_nn, W_dense, b_dense, W_out, b_out):
    N, D = h.shape
    DD = W_dense.shape[1]
    DO = W_out.shape[1]
    B = 1000

    def body(h_ref, a_ref, wn_ref, bn_ref, wd_ref, bd_ref, wo_ref, bo_ref,
             o_ref):
        s = h_ref[...] + a_ref[0] + a_ref[1]
        z = jnp.dot(s, wn_ref[...], preferred_element_type=jnp.float32) + bn_ref[...]
        z = jnp.where(z >= 0, z, 0.01 * z)
        z = jnp.dot(z, wd_ref[...], preferred_element_type=jnp.float32) + bd_ref[...]
        o_ref[...] = (
            jnp.dot(z, wo_ref[...], preferred_element_type=jnp.float32) + bo_ref[...]
        )

    return pl.pallas_call(
        body,
        grid=(N // B,),
        in_specs=[
            pl.BlockSpec((B, D), lambda i: (i, 0)),
            pl.BlockSpec((NC, B, D), lambda i: (0, i, 0)),
            pl.BlockSpec((D, D), lambda i: (0, 0)),
            pl.BlockSpec((1, D), lambda i: (0, 0)),
            pl.BlockSpec((D, DD), lambda i: (0, 0)),
            pl.BlockSpec((1, DD), lambda i: (0, 0)),
            pl.BlockSpec((DD, DO), lambda i: (0, 0)),
            pl.BlockSpec((1, DO), lambda i: (0, 0)),
        ],
        out_specs=pl.BlockSpec((B, DO), lambda i: (i, 0)),
        out_shape=jax.ShapeDtypeStruct((N, DO), jnp.float32),
    )(h, acc, W_nn, b_nn.reshape(1, D), W_dense, b_dense.reshape(1, DD),
      W_out, b_out.reshape(1, DO))


def kernel(x, edge_index, edge_attr, W_nn, b_nn, W_e0, b_e0, W_e1, b_e1,
           W_dense, b_dense, W_out, b_out):
    N = x.shape[0]
    E = edge_index.shape[1]
    EP = -(-E // (NC * NS * C * UNR)) * (NC * NS * C * UNR)
    pad = EP - E
    src2 = jnp.concatenate(
        [edge_index[0], jnp.zeros((pad,), jnp.int32)]).reshape(EP // C, C)
    dst2 = jnp.concatenate(
        [edge_index[1], jnp.full((pad,), N, jnp.int32)]).reshape(EP // C, C)
    t0 = _edge_linear(edge_attr, W_e0, b_e0, EP)
    t1 = _edge_linear(edge_attr, W_e1, b_e1, EP)
    acc0 = _sc_message_pass(x, src2, dst2, t0)
    h1 = _node_update(x, acc0, W_nn, b_nn)
    acc1 = _sc_message_pass(h1, src2, dst2, t1)
    return _final(h1, acc1, W_nn, b_nn, W_dense, b_dense, W_out, b_out)
